# baseline (speedup 1.0000x reference)
import jax
import jax.numpy as jnp
from jax import lax
from jax.experimental import pallas as pl
from jax.experimental.pallas import tpu as pltpu
from jax.experimental.pallas import tpu_sc as plsc

B = 16384
C = 64
N = 1000000
SLAB = 8

NC = 2
NS = 16
NW = NC * NS
B_PER_W = B // NW
DEPTH = 16


def _sc_gather(conf_hbm, idx_hbm, out_hbm, idx_v, slab_v, row_v, sems):
    wid = lax.axis_index("s") * NC + lax.axis_index("c")
    base = wid * B_PER_W
    pltpu.sync_copy(idx_hbm.at[pl.ds(base, B_PER_W)], idx_v)

    def fire(j):
        idx = idx_v[pl.ds(j, 1)][0]
        slab = (idx // SLAB) * SLAB
        buf = lax.rem(j, DEPTH)
        pltpu.async_copy(
            conf_hbm.at[pl.ds(slab, SLAB)],
            slab_v.at[pl.ds(buf * SLAB, SLAB)],
            sems.at[buf],
        )

    def retire(j):
        buf = lax.rem(j, DEPTH)
        pltpu.make_async_copy(
            conf_hbm.at[pl.ds(0, SLAB)],
            slab_v.at[pl.ds(buf * SLAB, SLAB)],
            sems.at[buf],
        ).wait()
        idx = idx_v[pl.ds(j, 1)][0]
        r = lax.rem(idx, SLAB)
        row_v[pl.ds(j, 1), :] = slab_v[pl.ds(buf * SLAB + r, 1), :]

    def body(j, _):
        @pl.when(j >= DEPTH)
        def _():
            retire(j - DEPTH)

        fire(j)
        return ()

    lax.fori_loop(0, B_PER_W, body, (), unroll=False)

    def drain(j, _):
        retire(j)
        return ()

    lax.fori_loop(B_PER_W - DEPTH, B_PER_W, drain, (), unroll=False)
    pltpu.sync_copy(row_v, out_hbm.at[pl.ds(base, B_PER_W)])


def _gather_rows(confidence, index):
    mesh = plsc.VectorSubcoreMesh(core_axis_name="c", subcore_axis_name="s")
    f = pl.kernel(
        _sc_gather,
        out_type=jax.ShapeDtypeStruct((B, C), jnp.float32),
        mesh=mesh,
        scratch_types=[
            pltpu.VMEM((B_PER_W,), jnp.int32),
            pltpu.VMEM((DEPTH * SLAB, C), jnp.float32),
            pltpu.VMEM((B_PER_W, C), jnp.float32),
            pltpu.SemaphoreType.DMA((DEPTH,)),
        ],
    )
    return f(confidence, index)


TC_BLK = 1024


def _tc_loss(x_ref, conf_ref, loss_ref, mean_ref):
    i = pl.program_id(0)
    x = x_ref[...]
    conf = conf_ref[...]
    m = jnp.max(x, axis=1, keepdims=True)
    lse = m + jnp.log(jnp.sum(jnp.exp(x - m), axis=1, keepdims=True))
    logsm = x - lse
    loss = -jnp.sum(logsm * conf, axis=1, keepdims=True)
    loss_ref[...] = loss

    @pl.when(i == 0)
    def _():
        mean_ref[...] = jnp.zeros_like(mean_ref)

    mean_ref[...] += jnp.sum(loss) * (1.0 / B)


def kernel(outputs, index, confidence):
    index = index.astype(jnp.int32)
    rows = _gather_rows(confidence, index)
    loss2d, mean2d = pl.pallas_call(
        _tc_loss,
        grid=(B // TC_BLK,),
        in_specs=[
            pl.BlockSpec((TC_BLK, C), lambda i: (i, 0)),
            pl.BlockSpec((TC_BLK, C), lambda i: (i, 0)),
        ],
        out_specs=[
            pl.BlockSpec((TC_BLK, 1), lambda i: (i, 0)),
            pl.BlockSpec((1, 1), lambda i: (0, 0)),
        ],
        out_shape=[
            jax.ShapeDtypeStruct((B, 1), jnp.float32),
            jax.ShapeDtypeStruct((1, 1), jnp.float32),
        ],
    )(outputs, rows)
    return (mean2d[0, 0], loss2d.reshape(B))


# speedup vs baseline: 1.1105x; 1.1105x over previous
import jax
import jax.numpy as jnp
from jax import lax
from jax.experimental import pallas as pl
from jax.experimental.pallas import tpu as pltpu
from jax.experimental.pallas import tpu_sc as plsc

B = 16384
C = 64
N = 1000000

NC = 2
NS = 16
NW = NC * NS
B_PER_W = B // NW
DEPTH = 32


def _sc_gather(conf_hbm, idx_hbm, out_hbm, idx_v, row_v, sem):
    wid = lax.axis_index("s") * NC + lax.axis_index("c")
    base = wid * B_PER_W
    pltpu.sync_copy(idx_hbm.at[pl.ds(base, B_PER_W)], idx_v)

    def fire(j):
        idx = idx_v[pl.ds(j, 1)][0]
        pltpu.async_copy(conf_hbm.at[pl.ds(idx, 1)], row_v.at[pl.ds(j, 1)], sem)

    def wait(j):
        pltpu.make_async_copy(
            conf_hbm.at[pl.ds(0, 1)], row_v.at[pl.ds(j, 1)], sem
        ).wait()

    def body(j, _):
        fire(j)

        @pl.when(j >= DEPTH)
        def _():
            wait(j - DEPTH)

        return ()

    lax.fori_loop(0, B_PER_W, body, (), unroll=False)

    def drain(j, _):
        wait(j)
        return ()

    lax.fori_loop(B_PER_W - DEPTH, B_PER_W, drain, (), unroll=False)
    pltpu.sync_copy(row_v, out_hbm.at[pl.ds(base, B_PER_W)])


def _gather_rows(confidence, index):
    mesh = plsc.VectorSubcoreMesh(core_axis_name="c", subcore_axis_name="s")
    f = pl.kernel(
        _sc_gather,
        out_type=jax.ShapeDtypeStruct((B, C), jnp.float32),
        mesh=mesh,
        scratch_types=[
            pltpu.VMEM((B_PER_W,), jnp.int32),
            pltpu.VMEM((B_PER_W, C), jnp.float32),
            pltpu.SemaphoreType.DMA,
        ],
    )
    return f(confidence, index)


TC_BLK = 1024


def _tc_loss(x_ref, conf_ref, loss_ref, mean_ref):
    i = pl.program_id(0)
    x = x_ref[...]
    conf = conf_ref[...]
    m = jnp.max(x, axis=1, keepdims=True)
    lse = m + jnp.log(jnp.sum(jnp.exp(x - m), axis=1, keepdims=True))
    logsm = x - lse
    loss = -jnp.sum(logsm * conf, axis=1, keepdims=True)
    loss_ref[...] = loss

    @pl.when(i == 0)
    def _():
        mean_ref[...] = jnp.zeros_like(mean_ref)

    mean_ref[...] += jnp.sum(loss) * (1.0 / B)


def kernel(outputs, index, confidence):
    index = index.astype(jnp.int32)
    rows = _gather_rows(confidence, index)
    loss2d, mean2d = pl.pallas_call(
        _tc_loss,
        grid=(B // TC_BLK,),
        in_specs=[
            pl.BlockSpec((TC_BLK, C), lambda i: (i, 0)),
            pl.BlockSpec((TC_BLK, C), lambda i: (i, 0)),
        ],
        out_specs=[
            pl.BlockSpec((TC_BLK, 1), lambda i: (i, 0)),
            pl.BlockSpec((1, 1), lambda i: (0, 0)),
        ],
        out_shape=[
            jax.ShapeDtypeStruct((B, 1), jnp.float32),
            jax.ShapeDtypeStruct((1, 1), jnp.float32),
        ],
    )(outputs, rows)
    return (mean2d[0, 0], loss2d.reshape(B))
